# baseline (speedup 1.0000x reference)
import functools

import jax
import jax.numpy as jnp
from jax.experimental import pallas as pl
from jax.experimental.pallas import tpu as pltpu

CLASSES = 128
SUPER = 8
CF = CLASSES // SUPER
BW = 2048

_F32 = jnp.float32


def _lrelu(v):
    return jnp.maximum(v, 0.01 * v)


def _mm(w, v):
    return jax.lax.dot_general(w, v, (((1,), (0,)), ((), ())),
                               preferred_element_type=_F32)


def _body(x_ref, wa_ref, ba_ref, wb_ref, cl2_b_ref, wc_ref, wd_ref,
          cl3_b_ref, we_ref, b2t_ref, xreal_ref, mask_ref):
    x = x_ref[0, :, 0, :]

    z1 = _lrelu(_mm(wa_ref[...], x) + ba_ref[...].reshape(256, 1))
    h1 = z1[0:128, :]
    r1 = z1[128:256, :]

    z2 = _mm(wb_ref[...], h1)
    h2 = _lrelu(z2[0:128, :] + cl2_b_ref[...].reshape(128, 1))
    y = z2[128:384, :] + _mm(wc_ref[...], r1)

    lg = _mm(wd_ref[...], h2) + cl3_b_ref[...].reshape(CLASSES + 1, 1)
    mask_ref[0, 0, 0, :] = _lrelu(lg[CLASSES, :])

    cls = lg[0:CLASSES, :]
    m = jnp.max(cls, axis=0, keepdims=True)
    row_iota = jax.lax.broadcasted_iota(jnp.int32, (CLASSES, BW), 0)
    inds = jnp.min(jnp.where(cls == m, row_iota, CLASSES), axis=0,
                   keepdims=True)

    s = inds // CF
    oh8 = (jax.lax.broadcasted_iota(jnp.int32, (SUPER, BW), 0)
           == s).astype(_F32)
    b32 = _mm(b2t_ref[...], oh8)
    x32 = y[0:32, :]
    for e in range(1, SUPER):
        x32 = jnp.where(s == e, y[e * 32:(e + 1) * 32, :], x32)
    x32 = _lrelu(x32 + b32)

    oh = (row_iota == inds).astype(_F32)
    sel = _mm(we_ref[...], oh)
    reg = (jnp.sum(x32 * sel[0:32, :], axis=0, keepdims=True) +
           sel[32:33, :])
    xreal_ref[0, 0, 0, :] = ((inds.astype(_F32) + reg) *
                             (1.0 / float(CLASSES)))[0, :]


@jax.jit
def _run(x_in, wa, ba, wb, cl2_b, wc, wd, cl3_b, we, b2t):
    B, C, H, W = x_in.shape
    grid = (B, W // BW)
    wspec = lambda shape: pl.BlockSpec(shape, lambda b, j: (0,) * len(shape))
    out_shapes = (
        jax.ShapeDtypeStruct((B, 1, 1, W), jnp.float32),
        jax.ShapeDtypeStruct((B, 1, 1, W), jnp.float32),
    )
    ospec = pl.BlockSpec((1, 1, 1, BW), lambda b, j: (b, 0, 0, j))
    return pl.pallas_call(
        _body,
        grid=grid,
        in_specs=[
            pl.BlockSpec((1, C, 1, BW), lambda b, j: (b, 0, 0, j)),
            wspec((256, 128)), wspec((256,)),
            wspec((384, 128)), wspec((128,)),
            wspec((256, 128)),
            wspec((CLASSES + 1, 128)), wspec((CLASSES + 1,)),
            wspec((33, 128)), wspec((32, SUPER)),
        ],
        out_specs=(ospec, ospec),
        out_shape=out_shapes,
    )(x_in, wa, ba, wb, cl2_b, wc, wd, cl3_b, we, b2t)


def kernel(x_in, cl1_w, cl1_b, cl2_w, cl2_b, cl3_w, cl3_b,
           reg1_w, reg1_b, reg2_w, reg2_b, reg3_w, reg3_b):
    w2all = jnp.transpose(reg2_w, (0, 2, 1)).reshape(SUPER * 32, 256)
    wa = jnp.concatenate([cl1_w, reg1_w], axis=0)
    ba = jnp.concatenate([cl1_b, reg1_b], axis=0)
    wb = jnp.concatenate([cl2_w, w2all[:, 128:256]], axis=0)
    wc = w2all[:, 0:128]
    we = jnp.concatenate([reg3_w[:, :, 0].T,
                          reg3_b[:, 0].reshape(1, CLASSES)], axis=0)
    b2t = reg2_b.T
    x_real, mask = _run(x_in, wa, ba, wb, cl2_b, wc, cl3_w, cl3_b, we, b2t)
    return (x_real, mask)


# speedup vs baseline: 1.1575x; 1.1575x over previous
import functools

import jax
import jax.numpy as jnp
from jax.experimental import pallas as pl
from jax.experimental.pallas import tpu as pltpu

CLASSES = 128
SUPER = 8
CF = CLASSES // SUPER
BW = 2048

_F32 = jnp.float32


def _lrelu(v):
    return jnp.maximum(v, 0.01 * v)


def _mm(w, v):
    return jax.lax.dot_general(w, v, (((1,), (0,)), ((), ())),
                               preferred_element_type=_F32)


def _body(x_ref, cl1_w_ref, cl1_b_ref, cl2_w_ref, cl2_b_ref, cl3_w_ref,
          cl3_b_ref, reg1_w_ref, reg1_b_ref, w2r_ref, w2h_ref, b2t_ref,
          we_ref, xreal_ref, mask_ref):
    x = x_ref[0, :, 0, :]

    h1 = _lrelu(_mm(cl1_w_ref[...], x) + cl1_b_ref[...].reshape(128, 1))
    h2 = _lrelu(_mm(cl2_w_ref[...], h1) + cl2_b_ref[...].reshape(128, 1))
    lg = _mm(cl3_w_ref[...], h2) + cl3_b_ref[...].reshape(CLASSES + 1, 1)
    mask_ref[0, 0, 0, :] = _lrelu(lg[CLASSES, :])

    cls = lg[0:CLASSES, :]
    m = jnp.max(cls, axis=0, keepdims=True)
    row_iota = jax.lax.broadcasted_iota(jnp.int32, (CLASSES, BW), 0)
    inds = jnp.min(jnp.where(cls == m, row_iota, CLASSES), axis=0,
                   keepdims=True)

    r1 = _lrelu(_mm(reg1_w_ref[...], x) + reg1_b_ref[...].reshape(128, 1))
    y = (_mm(w2r_ref[...], r1) +
         _mm(w2h_ref[...], h1))

    s = inds // CF
    oh8 = (jax.lax.broadcasted_iota(jnp.int32, (SUPER, BW), 0)
           == s).astype(_F32)
    b32 = _mm(b2t_ref[...], oh8)
    x32 = y[0:32, :]
    for e in range(1, SUPER):
        x32 = jnp.where(s == e, y[e * 32:(e + 1) * 32, :], x32)
    x32 = _lrelu(x32 + b32)

    oh = (row_iota == inds).astype(_F32)
    sel = _mm(we_ref[...], oh)
    reg = (jnp.sum(x32 * sel[0:32, :], axis=0, keepdims=True) +
           sel[32:33, :])
    xreal_ref[0, 0, 0, :] = ((inds.astype(_F32) + reg) *
                             (1.0 / float(CLASSES)))[0, :]


@jax.jit
def _run(x_in, cl1_w, cl1_b, cl2_w, cl2_b, cl3_w, cl3_b,
         reg1_w, reg1_b, w2r, w2h, b2t, we):
    B, C, H, W = x_in.shape
    grid = (B, W // BW)
    wspec = lambda shape: pl.BlockSpec(shape, lambda b, j: (0,) * len(shape))
    out_shapes = (
        jax.ShapeDtypeStruct((B, 1, 1, W), jnp.float32),
        jax.ShapeDtypeStruct((B, 1, 1, W), jnp.float32),
    )
    ospec = pl.BlockSpec((1, 1, 1, BW), lambda b, j: (b, 0, 0, j))
    return pl.pallas_call(
        _body,
        grid=grid,
        in_specs=[
            pl.BlockSpec((1, C, 1, BW), lambda b, j: (b, 0, 0, j)),
            wspec((128, 128)), wspec((128,)),
            wspec((128, 128)), wspec((128,)),
            wspec((CLASSES + 1, 128)), wspec((CLASSES + 1,)),
            wspec((128, 128)), wspec((128,)),
            wspec((256, 128)), wspec((256, 128)),
            wspec((32, SUPER)), wspec((33, 128)),
        ],
        out_specs=(ospec, ospec),
        out_shape=out_shapes,
    )(x_in, cl1_w, cl1_b, cl2_w, cl2_b, cl3_w, cl3_b,
      reg1_w, reg1_b, w2r, w2h, b2t, we)


def kernel(x_in, cl1_w, cl1_b, cl2_w, cl2_b, cl3_w, cl3_b,
           reg1_w, reg1_b, reg2_w, reg2_b, reg3_w, reg3_b):
    w2all = jnp.transpose(reg2_w, (0, 2, 1)).reshape(SUPER * 32, 256)
    w2r = w2all[:, 0:128]
    w2h = w2all[:, 128:256]
    b2t = reg2_b.T
    we = jnp.concatenate([reg3_w[:, :, 0].T,
                          reg3_b[:, 0].reshape(1, CLASSES)], axis=0)
    x_real, mask = _run(x_in, cl1_w, cl1_b, cl2_w, cl2_b, cl3_w, cl3_b,
                        reg1_w, reg1_b, w2r, w2h, b2t, we)
    return (x_real, mask)
